# baseline (speedup 1.0000x reference)
import functools

import jax
import jax.numpy as jnp
from jax import lax
from jax.experimental import pallas as pl
from jax.experimental.pallas import tpu as pltpu
from jax.experimental.pallas import tpu_sc as plsc

N = 10000
NPAD = 10240
E = 320000
F1 = 128
F2 = 64
NC = 2
NS = 16
NW = NC * NS
CH = 125
NROW = E // CH
NCH = NROW // NS
NBUF = 4
RPT = NPAD // NS
EPT = E // NW
NCH_D = 2 * EPT // CH
NBLK_D = NCH_D // NBUF
RPT_D = 2 * NPAD // NS


@functools.cache
def _mesh():
    return plsc.VectorSubcoreMesh(core_axis_name="c", subcore_axis_name="s",
                                  num_cores=NC, num_subcores=NS)


def _zero_vmem(ref, rows, width):
    zeros16 = jnp.zeros((16,), jnp.float32)

    def body(i, _):
        r = i // (width // 16)
        j = i % (width // 16)
        ref[r, pl.ds(j * 16, 16)] = zeros16
        return 0

    lax.fori_loop(0, rows * (width // 16), body, 0)


def _zero_vmem_1d(ref, n):
    zeros16 = jnp.zeros((16,), jnp.float32)

    def body(i, _):
        ref[pl.ds(i * 16, 16)] = zeros16
        return 0

    lax.fori_loop(0, n // 16, body, 0)


def _rsqrt16(x):
    i = plsc.bitcast(x, jnp.int32)
    i = 0x5F3759DF - lax.shift_right_arithmetic(i, 1)
    y = plsc.bitcast(i, jnp.float32)
    for _ in range(3):
        y = y * (1.5 - 0.5 * x * y * y)
    return y


def _norm_fill(nb, d0, d1):
    def body(k, _):
        d = d0[pl.ds(k * 16, 16)] + d1[pl.ds(k * 16, 16)]
        r = _rsqrt16(jnp.maximum(d, 1.0))
        nb[pl.ds(k * 16, 16)] = jnp.where(d > 0, r, 0.0)
        return 0

    lax.fori_loop(0, RPT // 16, body, 0)


@functools.cache
def _make_deg_kernel():
    return functools.partial(
        pl.kernel,
        out_type=jax.ShapeDtypeStruct((NC * 2 * NPAD,), jnp.float32),
        mesh=_mesh(),
        scratch_types=[
            pltpu.VMEM((NCH_D // 2, CH), jnp.int32),
            pltpu.VMEM((NCH_D // 2, CH), jnp.int32),
            pltpu.VMEM((CH,), jnp.float32),
            pltpu.VMEM((RPT_D,), jnp.float32),
            pltpu.VMEM_SHARED((NPAD,), jnp.float32),
            pltpu.VMEM_SHARED((NPAD,), jnp.float32),
            pltpu.SemaphoreType.DMA,
            pltpu.SemaphoreType.DMA,
            pltpu.SemaphoreType.DMA,
            pltpu.SemaphoreType.DMA,
        ],
        compiler_params=pltpu.CompilerParams(use_tc_tiling_on_sc=False),
    )(_deg_body)


def _deg_body(src_hbm, dst_hbm, deg_out, src_v, dst_v, ones_v, zero_v,
              dego_sh, degi_sh, *sems):
    cid = lax.axis_index("c")
    sid = lax.axis_index("s")
    wid = cid * NS + sid
    npair = NCH_D // 2

    pltpu.sync_copy(src_hbm.at[pl.ds(wid * npair, npair)], src_v)
    pltpu.sync_copy(dst_hbm.at[pl.ds(wid * npair, npair)], dst_v)

    ones16 = jnp.ones((16,), jnp.float32)

    def fill_ones(i, _):
        ones_v[pl.ds(i * 16, 16)] = ones16
        return 0

    lax.fori_loop(0, CH // 16, fill_ones, 0)
    ones_v[pl.ds(CH - 16, 16)] = ones16
    _zero_vmem_1d(zero_v, RPT_D)

    h = RPT_D // 2
    pltpu.sync_copy(zero_v.at[pl.ds(0, h)], dego_sh.at[pl.ds(sid * h, h)])
    pltpu.sync_copy(zero_v.at[pl.ds(0, h)], degi_sh.at[pl.ds(sid * h, h)])
    plsc.subcore_barrier()

    def scat(c, parity, b):
        if parity == 0:
            pltpu.async_copy(ones_v, dego_sh.at[src_v.at[c]], sems[b],
                             add=True)
        else:
            pltpu.async_copy(ones_v, degi_sh.at[dst_v.at[c]], sems[b],
                             add=True)

    def wait(b):
        pltpu.make_async_copy(ones_v, dego_sh.at[src_v.at[0]],
                              sems[b]).wait()

    for b in range(NBUF):
        scat(b // 2, b % 2, b)

    def blk_body(blk, _):
        for b in range(NBUF):
            wait(b)
            scat(blk * 2 + b // 2, b % 2, b)
        return 0

    lax.fori_loop(1, NBLK_D, blk_body, 0)
    for b in range(NBUF):
        wait(b)
    plsc.subcore_barrier()

    o = cid * 2 * NPAD + sid * h
    pltpu.sync_copy(dego_sh.at[pl.ds(sid * h, h)], deg_out.at[pl.ds(o, h)])
    pltpu.sync_copy(degi_sh.at[pl.ds(sid * h, h)],
                    deg_out.at[pl.ds(o + NPAD, h)])


SCH = 80


@functools.cache
def _make_edge_kernel(FH, mode):
    if FH == 32:
        nbuf, glag, slag = 8, 4, 4
    else:
        nbuf, glag, slag = 5, 3, 2
    assert nbuf == glag + slag and NCH % nbuf == 0
    nblk = NCH // nbuf

    out_type = [jax.ShapeDtypeStruct((NC * NPAD, FH), jnp.float32)]
    if mode == "pre":
        out_type += [jax.ShapeDtypeStruct((NPAD, FH), jnp.float32),
                     jax.ShapeDtypeStruct((NPAD, FH), jnp.float32)]

    @functools.partial(
        pl.kernel,
        out_type=out_type,
        mesh=_mesh(),
        scratch_types=(
            [pltpu.VMEM((NCH, CH), jnp.int32),
             pltpu.VMEM((NCH, CH), jnp.int32),
             pltpu.VMEM((RPT,), jnp.float32),
             pltpu.VMEM((RPT,), jnp.float32),
             pltpu.VMEM((FH,), jnp.float32)]
            + [pltpu.VMEM((CH, FH), jnp.float32) for _ in range(nbuf)]
            + [pltpu.VMEM((64, FH), jnp.float32),
               pltpu.VMEM_SHARED((NPAD, FH), jnp.float32)]
            + [pltpu.SemaphoreType.DMA for _ in range(2 * nbuf)]
        ),
        compiler_params=pltpu.CompilerParams(use_tc_tiling_on_sc=False,
                                             needs_layout_passes=False),
    )
    def _edge_kernel(yl_hbm, yr_hbm, src_hbm, dst_hbm, deg_hbm, b_hbm,
                     *rest):
        if mode == "pre":
            out_hbm, ysl_hbm, ysr_hbm = rest[:3]
            rest = rest[3:]
        else:
            out_hbm = rest[0]
            rest = rest[1:]
        src_v, dst_v, nb, db, bv = rest[:5]
        rows = list(rest[5:5 + nbuf])
        zslab = rest[5 + nbuf]
        acc_sh = rest[5 + nbuf + 1]
        sems = rest[5 + nbuf + 2:]
        sg = sems[:nbuf]
        ss = sems[nbuf:]
        cid = lax.axis_index("c")
        sid = lax.axis_index("s")
        base_r = sid * RPT

        pltpu.sync_copy(src_hbm.at[pl.ds(sid * NCH, NCH)], src_v)
        pltpu.sync_copy(dst_hbm.at[pl.ds(sid * NCH, NCH)], dst_v)

        doff = 0 if mode == "pre" else NPAD
        pltpu.sync_copy(deg_hbm.at[pl.ds(doff + base_r, RPT)], nb)
        pltpu.sync_copy(deg_hbm.at[pl.ds(2 * NPAD + doff + base_r, RPT)], db)
        _norm_fill(nb, nb, db)

        def scale_rows(buf, nrows, koff, biased):
            def gbody(g, _):
                nvec = nb[pl.ds(koff + g * 16, 16)]
                for j in range(16):
                    s = nvec[j]
                    r = g * 16 + j
                    for q in range(FH // 16):
                        v = buf[r, pl.ds(q * 16, 16)] * s
                        if biased:
                            v = v + bv[pl.ds(q * 16, 16)]
                        buf[r, pl.ds(q * 16, 16)] = v
                return 0

            lax.fori_loop(0, nrows // 16, gbody, 0)

        _zero_vmem(zslab, 64, FH)

        def zcopy(i, _):
            pltpu.sync_copy(zslab, acc_sh.at[pl.ds(base_r + i * 64, 64)])
            return 0

        lax.fori_loop(0, RPT // 64, zcopy, 0)

        if mode == "pre":
            def do_scale(y_hbm, ys_hbm):
                for k in range(RPT // SCH):
                    o = base_r + k * SCH
                    pltpu.sync_copy(y_hbm.at[pl.ds(o, SCH)],
                                    rows[0].at[pl.ds(0, SCH)])
                    scale_rows(rows[0], SCH, k * SCH, False)
                    pltpu.sync_copy(rows[0].at[pl.ds(0, SCH)],
                                    ys_hbm.at[pl.ds(o, SCH)])

            @pl.when(cid == 0)
            def _sl():
                do_scale(yl_hbm, ysl_hbm)

            @pl.when(cid == 1)
            def _sr():
                do_scale(yr_hbm, ysr_hbm)

            tl_hbm, tr_hbm = ysl_hbm, ysr_hbm
        else:
            @pl.when(cid == 0)
            def _b0():
                pltpu.sync_copy(b_hbm.at[0], bv)

            @pl.when(cid == 1)
            def _b1():
                pltpu.sync_copy(b_hbm.at[1], bv)

            tl_hbm, tr_hbm = yl_hbm, yr_hbm

        plsc.subcore_barrier()

        def gat(c, b):
            ix = src_v.at[c]

            @pl.when(cid == 0)
            def _l():
                pltpu.async_copy(tl_hbm.at[ix], rows[b], sg[b])

            @pl.when(cid == 1)
            def _r():
                pltpu.async_copy(tr_hbm.at[ix], rows[b], sg[b])

        def wait_gat(c, b):
            pltpu.make_async_copy(tl_hbm.at[src_v.at[c]], rows[b],
                                  sg[b]).wait()

        def scat(c, b):
            return pltpu.async_copy(rows[b], acc_sh.at[dst_v.at[c]], ss[b],
                                    add=True)

        def wait_scat(c, b):
            pltpu.make_async_copy(rows[b], acc_sh.at[dst_v.at[c]],
                                  ss[b]).wait()

        for j in range(glag):
            gat(j, j)

        def blk_body(blk, _):
            for b in range(nbuf):
                c = blk * nbuf + b
                wait_gat(c, b)
                scat(c, b)
                b2 = (b + glag) % nbuf

                @pl.when(c + glag < NCH)
                def _issue():
                    @pl.when(c >= slag)
                    def _drain():
                        wait_scat(c - slag, b2)

                    gat(c + glag, b2)
            return 0

        lax.fori_loop(0, nblk, blk_body, 0)
        for b in range(nbuf):
            wait_scat(0, b)
        plsc.subcore_barrier()

        o = cid * NPAD + base_r
        if mode == "pre":
            pltpu.sync_copy(acc_sh.at[pl.ds(base_r, RPT)],
                            out_hbm.at[pl.ds(o, RPT)])
        else:
            for k in range(RPT // SCH):
                pltpu.sync_copy(acc_sh.at[pl.ds(base_r + k * SCH, SCH)],
                                rows[0].at[pl.ds(0, SCH)])
                scale_rows(rows[0], SCH, k * SCH, True)
                pltpu.sync_copy(rows[0].at[pl.ds(0, SCH)],
                                out_hbm.at[pl.ds(o + k * SCH, SCH)])

    return _edge_kernel


def _norms(degs_ref):
    dego = degs_ref[:, 0:1] + degs_ref[:, 2:3]
    degi = degs_ref[:, 1:2] + degs_ref[:, 3:4]
    nsrc = jnp.where(dego > 0, lax.rsqrt(jnp.maximum(dego, 1.0)), 0.0)
    ndst = jnp.where(degi > 0, lax.rsqrt(jnp.maximum(degi, 1.0)), 0.0)
    return nsrc, ndst


def _tc0_body(x_ref, w1_ref, yl_ref, yr_ref):
    y = jnp.dot(x_ref[...], w1_ref[...], preferred_element_type=jnp.float32)
    yl_ref[...] = y[:, :F1 // 2]
    yr_ref[...] = y[:, F1 // 2:]


def _tc2_body(a_ref, degs_ref, w2_ref, b1_ref, yl_ref, yr_ref):
    nsrc, ndst = _norms(degs_ref)
    a = jnp.concatenate([a_ref[0], a_ref[1]], axis=1)
    h = jax.nn.relu(a * ndst + b1_ref[...])
    y = jnp.dot(h, w2_ref[...],
                preferred_element_type=jnp.float32) * nsrc
    yl_ref[...] = y[:N, :F2 // 2]
    yr_ref[...] = y[:N, F2 // 2:]


_tc0 = pl.pallas_call(
    _tc0_body, out_shape=[
        jax.ShapeDtypeStruct((NPAD, F1 // 2), jnp.float32),
        jax.ShapeDtypeStruct((NPAD, F1 // 2), jnp.float32)])
_tc2 = pl.pallas_call(
    _tc2_body, out_shape=[
        jax.ShapeDtypeStruct((N, F2 // 2), jnp.float32),
        jax.ShapeDtypeStruct((N, F2 // 2), jnp.float32)])


def kernel(features, edge_index, W1, b1, W2, b2):
    src = edge_index[0].astype(jnp.int32)
    dst = edge_index[1].astype(jnp.int32)
    src2 = src.reshape(NROW, CH)
    dst2 = dst.reshape(NROW, CH)
    xp = jnp.pad(features, ((0, NPAD - N), (0, 0)))

    deg_f = _make_deg_kernel()(src2, dst2)
    degs = deg_f.reshape(NC * 2, NPAD).T

    y1l, y1r = _tc0(xp, W1)
    zb = jnp.zeros((NC, F1 // 2), jnp.float32)
    a1, _, _ = _make_edge_kernel(F1 // 2, "pre")(
        y1l, y1r, src2, dst2, deg_f, zb)
    a1 = a1.reshape(NC, NPAD, F1 // 2)
    y2l, y2r = _tc2(a1, degs, W2, b1.reshape(1, F1))
    a2, = _make_edge_kernel(F2 // 2, "post")(
        y2l, y2r, src2, dst2, deg_f, b2.reshape(NC, F2 // 2))
    a2 = a2.reshape(NC, NPAD, F2 // 2)
    return jnp.concatenate([a2[0, :N], a2[1, :N]], axis=1)


# speedup vs baseline: 1.1145x; 1.1145x over previous
import functools

import jax
import jax.numpy as jnp
from jax import lax
from jax.experimental import pallas as pl
from jax.experimental.pallas import tpu as pltpu
from jax.experimental.pallas import tpu_sc as plsc

N = 10000
NPAD = 10240
E = 320000
F1 = 128
F2 = 64
NC = 2
NS = 16
NW = NC * NS
CH = 125
NROW = E // CH
NCH = NROW // NS
NBUF = 4
RPT = NPAD // NS
EPT = E // NW
NCH_D = 2 * EPT // CH
NBLK_D = NCH_D // NBUF
RPT_D = 2 * NPAD // NS


@functools.cache
def _mesh():
    return plsc.VectorSubcoreMesh(core_axis_name="c", subcore_axis_name="s",
                                  num_cores=NC, num_subcores=NS)


def _zero_vmem(ref, rows, width):
    zeros16 = jnp.zeros((16,), jnp.float32)

    def body(i, _):
        r = i // (width // 16)
        j = i % (width // 16)
        ref[r, pl.ds(j * 16, 16)] = zeros16
        return 0

    lax.fori_loop(0, rows * (width // 16), body, 0)


def _zero_vmem_1d(ref, n):
    zeros16 = jnp.zeros((16,), jnp.float32)

    def body(i, _):
        ref[pl.ds(i * 16, 16)] = zeros16
        return 0

    lax.fori_loop(0, n // 16, body, 0)


def _rsqrt16(x):
    i = plsc.bitcast(x, jnp.int32)
    i = 0x5F3759DF - lax.shift_right_arithmetic(i, 1)
    y = plsc.bitcast(i, jnp.float32)
    for _ in range(3):
        y = y * (1.5 - 0.5 * x * y * y)
    return y


def _norm_fill(nb, d0, d1):
    def body(k, _):
        d = d0[pl.ds(k * 16, 16)] + d1[pl.ds(k * 16, 16)]
        r = _rsqrt16(jnp.maximum(d, 1.0))
        nb[pl.ds(k * 16, 16)] = jnp.where(d > 0, r, 0.0)
        return 0

    lax.fori_loop(0, RPT // 16, body, 0)


@functools.cache
def _make_deg_kernel():
    return functools.partial(
        pl.kernel,
        out_type=jax.ShapeDtypeStruct((NC * 2 * NPAD,), jnp.float32),
        mesh=_mesh(),
        scratch_types=[
            pltpu.VMEM((NCH_D // 2, CH), jnp.int32),
            pltpu.VMEM((NCH_D // 2, CH), jnp.int32),
            pltpu.VMEM((CH,), jnp.float32),
            pltpu.VMEM((RPT_D,), jnp.float32),
            pltpu.VMEM_SHARED((NPAD,), jnp.float32),
            pltpu.VMEM_SHARED((NPAD,), jnp.float32),
            pltpu.SemaphoreType.DMA,
            pltpu.SemaphoreType.DMA,
            pltpu.SemaphoreType.DMA,
            pltpu.SemaphoreType.DMA,
        ],
        compiler_params=pltpu.CompilerParams(use_tc_tiling_on_sc=False),
    )(_deg_body)


def _deg_body(src_hbm, dst_hbm, deg_out, src_v, dst_v, ones_v, zero_v,
              dego_sh, degi_sh, *sems):
    cid = lax.axis_index("c")
    sid = lax.axis_index("s")
    wid = cid * NS + sid
    npair = NCH_D // 2

    pltpu.sync_copy(src_hbm.at[pl.ds(wid * npair, npair)], src_v)
    pltpu.sync_copy(dst_hbm.at[pl.ds(wid * npair, npair)], dst_v)

    ones16 = jnp.ones((16,), jnp.float32)

    def fill_ones(i, _):
        ones_v[pl.ds(i * 16, 16)] = ones16
        return 0

    lax.fori_loop(0, CH // 16, fill_ones, 0)
    ones_v[pl.ds(CH - 16, 16)] = ones16
    _zero_vmem_1d(zero_v, RPT_D)

    h = RPT_D // 2
    pltpu.sync_copy(zero_v.at[pl.ds(0, h)], dego_sh.at[pl.ds(sid * h, h)])
    pltpu.sync_copy(zero_v.at[pl.ds(0, h)], degi_sh.at[pl.ds(sid * h, h)])
    plsc.subcore_barrier()

    def scat(c, parity, b):
        if parity == 0:
            pltpu.async_copy(ones_v, dego_sh.at[src_v.at[c]], sems[b],
                             add=True)
        else:
            pltpu.async_copy(ones_v, degi_sh.at[dst_v.at[c]], sems[b],
                             add=True)

    def wait(b):
        pltpu.make_async_copy(ones_v, dego_sh.at[src_v.at[0]],
                              sems[b]).wait()

    for b in range(NBUF):
        scat(b // 2, b % 2, b)

    def blk_body(blk, _):
        for b in range(NBUF):
            wait(b)
            scat(blk * 2 + b // 2, b % 2, b)
        return 0

    lax.fori_loop(1, NBLK_D, blk_body, 0)
    for b in range(NBUF):
        wait(b)
    plsc.subcore_barrier()

    o = cid * 2 * NPAD + sid * h
    pltpu.sync_copy(dego_sh.at[pl.ds(sid * h, h)], deg_out.at[pl.ds(o, h)])
    pltpu.sync_copy(degi_sh.at[pl.ds(sid * h, h)],
                    deg_out.at[pl.ds(o + NPAD, h)])


@functools.cache
def _make_edge_kernel(FH):
    if FH == 32:
        nbuf, glag, slag = 8, 4, 4
    else:
        nbuf, glag, slag = 6, 4, 2
    assert nbuf == glag + slag
    nblk = NCH // nbuf
    ntail = NCH - nblk * nbuf

    @functools.partial(
        pl.kernel,
        out_type=jax.ShapeDtypeStruct((NC * NPAD, FH), jnp.float32),
        mesh=_mesh(),
        scratch_types=(
            [pltpu.VMEM((NCH, CH), jnp.int32),
             pltpu.VMEM((NCH, CH), jnp.int32)]
            + [pltpu.VMEM((CH, FH), jnp.float32) for _ in range(nbuf)]
            + [pltpu.VMEM_SHARED((NPAD, FH), jnp.float32)]
            + [pltpu.SemaphoreType.DMA for _ in range(2 * nbuf)]
        ),
        compiler_params=pltpu.CompilerParams(use_tc_tiling_on_sc=False),
    )
    def _edge_kernel(yl_hbm, yr_hbm, src_hbm, dst_hbm, out_hbm,
                     src_v, dst_v, *rest):
        rows = list(rest[:nbuf])
        acc_sh = rest[nbuf]
        sems = rest[nbuf + 1:]
        sg = sems[:nbuf]
        ss = sems[nbuf:]
        cid = lax.axis_index("c")
        sid = lax.axis_index("s")
        base_r = sid * RPT

        pltpu.sync_copy(src_hbm.at[pl.ds(sid * NCH, NCH)], src_v)
        pltpu.sync_copy(dst_hbm.at[pl.ds(sid * NCH, NCH)], dst_v)

        _zero_vmem(rows[0], CH, FH)

        def zcopy(i, _):
            pltpu.sync_copy(rows[0], acc_sh.at[pl.ds(base_r + i * CH, CH)])
            return 0

        lax.fori_loop(0, RPT // CH, zcopy, 0)
        pltpu.sync_copy(rows[0].at[pl.ds(0, RPT - (RPT // CH) * CH)],
                        acc_sh.at[pl.ds(base_r + (RPT // CH) * CH,
                                        RPT - (RPT // CH) * CH)])
        plsc.subcore_barrier()

        tl_hbm, tr_hbm = yl_hbm, yr_hbm

        def gat(c, b):
            ix = src_v.at[c]

            @pl.when(cid == 0)
            def _l():
                pltpu.async_copy(tl_hbm.at[ix], rows[b], sg[b])

            @pl.when(cid == 1)
            def _r():
                pltpu.async_copy(tr_hbm.at[ix], rows[b], sg[b])

        def wait_gat(c, b):
            pltpu.make_async_copy(tl_hbm.at[src_v.at[c]], rows[b],
                                  sg[b]).wait()

        def scat(c, b):
            return pltpu.async_copy(rows[b], acc_sh.at[dst_v.at[c]], ss[b],
                                    add=True)

        def wait_scat(c, b):
            pltpu.make_async_copy(rows[b], acc_sh.at[dst_v.at[c]],
                                  ss[b]).wait()

        for j in range(glag):
            gat(j, j)

        def blk_body(blk, _):
            for b in range(nbuf):
                c = blk * nbuf + b
                wait_gat(c, b)
                scat(c, b)
                b2 = (b + glag) % nbuf

                @pl.when(c + glag < NCH)
                def _issue():
                    @pl.when(c >= slag)
                    def _drain():
                        wait_scat(c - slag, b2)

                    gat(c + glag, b2)
            return 0

        lax.fori_loop(0, nblk, blk_body, 0)
        for j in range(ntail):
            c = nblk * nbuf + j
            b = c % nbuf
            wait_gat(c, b)
            scat(c, b)
        for b in range(nbuf):
            wait_scat(0, b)
        plsc.subcore_barrier()

        o = cid * NPAD + base_r
        pltpu.sync_copy(acc_sh.at[pl.ds(base_r, RPT)],
                        out_hbm.at[pl.ds(o, RPT)])

    return _edge_kernel


def _norms(degs_ref):
    dego = degs_ref[:, 0:1] + degs_ref[:, 2:3]
    degi = degs_ref[:, 1:2] + degs_ref[:, 3:4]
    nsrc = jnp.where(dego > 0, lax.rsqrt(jnp.maximum(dego, 1.0)), 0.0)
    ndst = jnp.where(degi > 0, lax.rsqrt(jnp.maximum(degi, 1.0)), 0.0)
    return nsrc, ndst


def _tc0_body(x_ref, w1_ref, y_ref):
    y_ref[...] = jnp.dot(x_ref[...], w1_ref[...],
                         preferred_element_type=jnp.float32)


def _tc1s_body(y_ref, degs_ref, yl_ref, yr_ref):
    nsrc, _ = _norms(degs_ref)
    y = y_ref[...] * nsrc[:N]
    yl_ref[...] = y[:, :F1 // 2]
    yr_ref[...] = y[:, F1 // 2:]


def _tc2_body(a_ref, degs_ref, w2_ref, b1_ref, yl_ref, yr_ref):
    nsrc, ndst = _norms(degs_ref)
    a = jnp.concatenate([a_ref[0], a_ref[1]], axis=1)
    h = jax.nn.relu(a * ndst + b1_ref[...])
    y = jnp.dot(h, w2_ref[...],
                preferred_element_type=jnp.float32) * nsrc
    yl_ref[...] = y[:N, :F2 // 2]
    yr_ref[...] = y[:N, F2 // 2:]


def _tc3_body(a_ref, degs_ref, b2_ref, y_ref):
    _, ndst = _norms(degs_ref)
    a = jnp.concatenate([a_ref[0], a_ref[1]], axis=1)
    y_ref[...] = (a * ndst + b2_ref[...])[:N]


_tc0 = pl.pallas_call(
    _tc0_body, out_shape=jax.ShapeDtypeStruct((N, F1), jnp.float32))
_tc1s = pl.pallas_call(
    _tc1s_body, out_shape=[
        jax.ShapeDtypeStruct((N, F1 // 2), jnp.float32),
        jax.ShapeDtypeStruct((N, F1 // 2), jnp.float32)])
_tc2 = pl.pallas_call(
    _tc2_body, out_shape=[
        jax.ShapeDtypeStruct((N, F2 // 2), jnp.float32),
        jax.ShapeDtypeStruct((N, F2 // 2), jnp.float32)])
_tc3 = pl.pallas_call(
    _tc3_body, out_shape=jax.ShapeDtypeStruct((N, F2), jnp.float32))


def kernel(features, edge_index, W1, b1, W2, b2):
    src = edge_index[0].astype(jnp.int32)
    dst = edge_index[1].astype(jnp.int32)
    src2 = src.reshape(NROW, CH)
    dst2 = dst.reshape(NROW, CH)

    deg_f = _make_deg_kernel()(src2, dst2)
    degs = deg_f.reshape(NC * 2, NPAD).T

    y1 = _tc0(features, W1)
    y1l, y1r = _tc1s(y1, degs)
    a1 = _make_edge_kernel(F1 // 2)(y1l, y1r, src2, dst2)
    a1 = a1.reshape(NC, NPAD, F1 // 2)
    y2l, y2r = _tc2(a1, degs, W2, b1.reshape(1, F1))
    a2 = _make_edge_kernel(F2 // 2)(y2l, y2r, src2, dst2)
    a2 = a2.reshape(NC, NPAD, F2 // 2)
    out = _tc3(a2, degs, b2.reshape(1, F2))
    return out


# speedup vs baseline: 1.1337x; 1.0172x over previous
import functools

import jax
import jax.numpy as jnp
from jax import lax
from jax.experimental import pallas as pl
from jax.experimental.pallas import tpu as pltpu
from jax.experimental.pallas import tpu_sc as plsc

N = 10000
NPAD = 10240
E = 320000
F1 = 128
F2 = 64
NC = 2
NS = 16
NW = NC * NS
CH = 125
NROW = E // CH
NCH = NROW // NS
NBUF = 4
RPT = NPAD // NS
EPT = E // NW
NCH_D = 2 * EPT // CH
NBLK_D = NCH_D // NBUF
RPT_D = 2 * NPAD // NS


@functools.cache
def _mesh():
    return plsc.VectorSubcoreMesh(core_axis_name="c", subcore_axis_name="s",
                                  num_cores=NC, num_subcores=NS)


def _zero_vmem(ref, rows, width):
    zeros16 = jnp.zeros((16,), jnp.float32)

    def body(i, _):
        r = i // (width // 16)
        j = i % (width // 16)
        ref[r, pl.ds(j * 16, 16)] = zeros16
        return 0

    lax.fori_loop(0, rows * (width // 16), body, 0)


def _zero_vmem_1d(ref, n):
    zeros16 = jnp.zeros((16,), jnp.float32)

    def body(i, _):
        ref[pl.ds(i * 16, 16)] = zeros16
        return 0

    lax.fori_loop(0, n // 16, body, 0)


def _rsqrt16(x):
    i = plsc.bitcast(x, jnp.int32)
    i = 0x5F3759DF - lax.shift_right_arithmetic(i, 1)
    y = plsc.bitcast(i, jnp.float32)
    for _ in range(3):
        y = y * (1.5 - 0.5 * x * y * y)
    return y


def _norm_fill(nb, d0, d1):
    def body(k, _):
        d = d0[pl.ds(k * 16, 16)] + d1[pl.ds(k * 16, 16)]
        r = _rsqrt16(jnp.maximum(d, 1.0))
        nb[pl.ds(k * 16, 16)] = jnp.where(d > 0, r, 0.0)
        return 0

    lax.fori_loop(0, RPT // 16, body, 0)


@functools.cache
def _make_deg_kernel():
    return functools.partial(
        pl.kernel,
        out_type=jax.ShapeDtypeStruct((NC * 2 * NPAD,), jnp.float32),
        mesh=_mesh(),
        scratch_types=[
            pltpu.VMEM((NCH_D // 2, CH), jnp.int32),
            pltpu.VMEM((NCH_D // 2, CH), jnp.int32),
            pltpu.VMEM((CH,), jnp.float32),
            pltpu.VMEM((RPT_D,), jnp.float32),
            pltpu.VMEM_SHARED((NPAD,), jnp.float32),
            pltpu.VMEM_SHARED((NPAD,), jnp.float32),
            pltpu.SemaphoreType.DMA,
            pltpu.SemaphoreType.DMA,
            pltpu.SemaphoreType.DMA,
            pltpu.SemaphoreType.DMA,
        ],
        compiler_params=pltpu.CompilerParams(use_tc_tiling_on_sc=False),
    )(_deg_body)


def _deg_body(src_hbm, dst_hbm, deg_out, src_v, dst_v, ones_v, zero_v,
              dego_sh, degi_sh, *sems):
    cid = lax.axis_index("c")
    sid = lax.axis_index("s")
    wid = cid * NS + sid
    npair = NCH_D // 2

    pltpu.sync_copy(src_hbm.at[pl.ds(wid * npair, npair)], src_v)
    pltpu.sync_copy(dst_hbm.at[pl.ds(wid * npair, npair)], dst_v)

    ones16 = jnp.ones((16,), jnp.float32)

    def fill_ones(i, _):
        ones_v[pl.ds(i * 16, 16)] = ones16
        return 0

    lax.fori_loop(0, CH // 16, fill_ones, 0)
    ones_v[pl.ds(CH - 16, 16)] = ones16
    _zero_vmem_1d(zero_v, RPT_D)

    h = RPT_D // 2
    pltpu.sync_copy(zero_v.at[pl.ds(0, h)], dego_sh.at[pl.ds(sid * h, h)])
    pltpu.sync_copy(zero_v.at[pl.ds(0, h)], degi_sh.at[pl.ds(sid * h, h)])
    plsc.subcore_barrier()

    def scat(c, parity, b):
        if parity == 0:
            pltpu.async_copy(ones_v, dego_sh.at[src_v.at[c]], sems[b],
                             add=True)
        else:
            pltpu.async_copy(ones_v, degi_sh.at[dst_v.at[c]], sems[b],
                             add=True)

    def wait(b):
        pltpu.make_async_copy(ones_v, dego_sh.at[src_v.at[0]],
                              sems[b]).wait()

    for b in range(NBUF):
        scat(b // 2, b % 2, b)

    def blk_body(blk, _):
        for b in range(NBUF):
            wait(b)
            scat(blk * 2 + b // 2, b % 2, b)
        return 0

    lax.fori_loop(1, NBLK_D, blk_body, 0)
    for b in range(NBUF):
        wait(b)
    plsc.subcore_barrier()

    o = cid * 2 * NPAD + sid * h
    pltpu.sync_copy(dego_sh.at[pl.ds(sid * h, h)], deg_out.at[pl.ds(o, h)])
    pltpu.sync_copy(degi_sh.at[pl.ds(sid * h, h)],
                    deg_out.at[pl.ds(o + NPAD, h)])


@functools.cache
def _make_edge_kernel(FH):
    if FH == 32:
        nbuf, glag, slag = 10, 6, 4
    else:
        nbuf, glag, slag = 6, 4, 2
    assert nbuf == glag + slag
    nblk = NCH // nbuf
    ntail = NCH - nblk * nbuf

    @functools.partial(
        pl.kernel,
        out_type=jax.ShapeDtypeStruct((NC * NPAD, FH), jnp.float32),
        mesh=_mesh(),
        scratch_types=(
            [pltpu.VMEM((NCH, CH), jnp.int32),
             pltpu.VMEM((NCH, CH), jnp.int32)]
            + [pltpu.VMEM((CH, FH), jnp.float32) for _ in range(nbuf)]
            + [pltpu.VMEM_SHARED((NPAD, FH), jnp.float32)]
            + [pltpu.SemaphoreType.DMA for _ in range(2 * nbuf)]
        ),
        compiler_params=pltpu.CompilerParams(use_tc_tiling_on_sc=False),
    )
    def _edge_kernel(yl_hbm, yr_hbm, src_hbm, dst_hbm, out_hbm,
                     src_v, dst_v, *rest):
        rows = list(rest[:nbuf])
        acc_sh = rest[nbuf]
        sems = rest[nbuf + 1:]
        sg = sems[:nbuf]
        ss = sems[nbuf:]
        cid = lax.axis_index("c")
        sid = lax.axis_index("s")
        base_r = sid * RPT

        pltpu.sync_copy(src_hbm.at[pl.ds(sid * NCH, NCH)], src_v)
        pltpu.sync_copy(dst_hbm.at[pl.ds(sid * NCH, NCH)], dst_v)

        _zero_vmem(rows[0], CH, FH)

        def zcopy(i, _):
            pltpu.sync_copy(rows[0], acc_sh.at[pl.ds(base_r + i * CH, CH)])
            return 0

        lax.fori_loop(0, RPT // CH, zcopy, 0)
        pltpu.sync_copy(rows[0].at[pl.ds(0, RPT - (RPT // CH) * CH)],
                        acc_sh.at[pl.ds(base_r + (RPT // CH) * CH,
                                        RPT - (RPT // CH) * CH)])
        plsc.subcore_barrier()

        tl_hbm, tr_hbm = yl_hbm, yr_hbm

        def gat(c, b):
            ix = src_v.at[c]

            @pl.when(cid == 0)
            def _l():
                pltpu.async_copy(tl_hbm.at[ix], rows[b], sg[b])

            @pl.when(cid == 1)
            def _r():
                pltpu.async_copy(tr_hbm.at[ix], rows[b], sg[b])

        def wait_gat(c, b):
            pltpu.make_async_copy(tl_hbm.at[src_v.at[c]], rows[b],
                                  sg[b]).wait()

        def scat(c, b):
            return pltpu.async_copy(rows[b], acc_sh.at[dst_v.at[c]], ss[b],
                                    add=True)

        def wait_scat(c, b):
            pltpu.make_async_copy(rows[b], acc_sh.at[dst_v.at[c]],
                                  ss[b]).wait()

        for j in range(glag):
            gat(j, j)

        def blk_body(blk, _):
            for b in range(nbuf):
                c = blk * nbuf + b
                wait_gat(c, b)
                scat(c, b)
                b2 = (b + glag) % nbuf

                @pl.when(c + glag < NCH)
                def _issue():
                    @pl.when(c >= slag)
                    def _drain():
                        wait_scat(c - slag, b2)

                    gat(c + glag, b2)
            return 0

        lax.fori_loop(0, nblk, blk_body, 0)
        for j in range(ntail):
            c = nblk * nbuf + j
            b = c % nbuf
            wait_gat(c, b)
            scat(c, b)
        for b in range(nbuf):
            wait_scat(0, b)
        plsc.subcore_barrier()

        o = cid * NPAD + base_r
        pltpu.sync_copy(acc_sh.at[pl.ds(base_r, RPT)],
                        out_hbm.at[pl.ds(o, RPT)])

    return _edge_kernel


def _norms(degs_ref):
    dego = degs_ref[:, 0:1] + degs_ref[:, 2:3]
    degi = degs_ref[:, 1:2] + degs_ref[:, 3:4]
    nsrc = jnp.where(dego > 0, lax.rsqrt(jnp.maximum(dego, 1.0)), 0.0)
    ndst = jnp.where(degi > 0, lax.rsqrt(jnp.maximum(degi, 1.0)), 0.0)
    return nsrc, ndst


def _tc0_body(x_ref, w1_ref, y_ref):
    y_ref[...] = jnp.dot(x_ref[...], w1_ref[...],
                         preferred_element_type=jnp.float32)


def _tc1s_body(y_ref, degs_ref, yl_ref, yr_ref):
    nsrc, _ = _norms(degs_ref)
    y = y_ref[...] * nsrc[:N]
    yl_ref[...] = y[:, :F1 // 2]
    yr_ref[...] = y[:, F1 // 2:]


def _tc2_body(a_ref, degs_ref, w2_ref, b1_ref, yl_ref, yr_ref):
    nsrc, ndst = _norms(degs_ref)
    a = jnp.concatenate([a_ref[0], a_ref[1]], axis=1)
    h = jax.nn.relu(a * ndst + b1_ref[...])
    y = jnp.dot(h, w2_ref[...],
                preferred_element_type=jnp.float32) * nsrc
    yl_ref[...] = y[:N, :F2 // 2]
    yr_ref[...] = y[:N, F2 // 2:]


def _tc3_body(a_ref, degs_ref, b2_ref, y_ref):
    _, ndst = _norms(degs_ref)
    a = jnp.concatenate([a_ref[0], a_ref[1]], axis=1)
    y_ref[...] = (a * ndst + b2_ref[...])[:N]


_tc0 = pl.pallas_call(
    _tc0_body, out_shape=jax.ShapeDtypeStruct((N, F1), jnp.float32))
_tc1s = pl.pallas_call(
    _tc1s_body, out_shape=[
        jax.ShapeDtypeStruct((N, F1 // 2), jnp.float32),
        jax.ShapeDtypeStruct((N, F1 // 2), jnp.float32)])
_tc2 = pl.pallas_call(
    _tc2_body, out_shape=[
        jax.ShapeDtypeStruct((N, F2 // 2), jnp.float32),
        jax.ShapeDtypeStruct((N, F2 // 2), jnp.float32)])
_tc3 = pl.pallas_call(
    _tc3_body, out_shape=jax.ShapeDtypeStruct((N, F2), jnp.float32))


def kernel(features, edge_index, W1, b1, W2, b2):
    src = edge_index[0].astype(jnp.int32)
    dst = edge_index[1].astype(jnp.int32)
    src2 = src.reshape(NROW, CH)
    dst2 = dst.reshape(NROW, CH)

    deg_f = _make_deg_kernel()(src2, dst2)
    degs = deg_f.reshape(NC * 2, NPAD).T

    y1 = _tc0(features, W1)
    y1l, y1r = _tc1s(y1, degs)
    a1 = _make_edge_kernel(F1 // 2)(y1l, y1r, src2, dst2)
    a1 = a1.reshape(NC, NPAD, F1 // 2)
    y2l, y2r = _tc2(a1, degs, W2, b1.reshape(1, F1))
    a2 = _make_edge_kernel(F2 // 2)(y2l, y2r, src2, dst2)
    a2 = a2.reshape(NC, NPAD, F2 // 2)
    out = _tc3(a2, degs, b2.reshape(1, F2))
    return out
